# Initial kernel scaffold; baseline (speedup 1.0000x reference)
import jax
import jax.numpy as jnp
from jax.experimental import pallas as pl


def kernel(x, tok_table, pos_table):
    raise NotImplementedError("write your pallas kernel here")


# speedup vs baseline: 1.2061x; 1.2061x over previous
import functools

import jax
import jax.numpy as jnp
from jax import lax
from jax.experimental import pallas as pl
from jax.experimental.pallas import tpu as pltpu
from jax.experimental.pallas import tpu_sc as plsc


def _build_kernel(B, S, V, E):
    info = plsc.get_sparse_core_info()
    NC, NS, L = info.num_cores, info.num_subcores, info.num_lanes
    NW = NC * NS
    assert S % NW == 0
    s_per_w = S // NW
    CH = min(16, s_per_w)
    assert s_per_w % CH == 0 and E % L == 0
    NCH = s_per_w // CH
    NSTEPS = NCH * B
    NBUF = min(4, NSTEPS)
    NPOS = min(2, NCH)

    mesh = plsc.VectorSubcoreMesh(core_axis_name="c", subcore_axis_name="s")

    scratch = [pltpu.VMEM((B * s_per_w,), jnp.int32)]
    scratch += [pltpu.VMEM((CH, E), jnp.float32) for _ in range(NBUF)]
    scratch += [pltpu.VMEM((CH, E), jnp.float32) for _ in range(NPOS)]
    scratch += [pltpu.SemaphoreType.DMA for _ in range(2 * NBUF + NPOS)]

    @functools.partial(
        pl.kernel,
        mesh=mesh,
        out_type=jax.ShapeDtypeStruct((B, S, E), jnp.float32),
        scratch_types=scratch,
    )
    def emb_kernel(x_hbm, tok_hbm, pos_hbm, out_hbm, idx_v, *rest):
        tok = list(rest[:NBUF])
        posb = list(rest[NBUF:NBUF + NPOS])
        sems = rest[NBUF + NPOS:]
        gsem = list(sems[:NBUF])
        osem = list(sems[NBUF:2 * NBUF])
        psem = list(sems[2 * NBUF:])

        wid = lax.axis_index("s") * NC + lax.axis_index("c")
        s_base = wid * s_per_w

        for b in range(B):
            pltpu.sync_copy(
                x_hbm.at[b, pl.ds(s_base, s_per_w)],
                idx_v.at[pl.ds(b * s_per_w, s_per_w)],
            )

        def start_gather(t):
            c, b = divmod(t, B)
            k = t % NBUF
            return pltpu.async_copy(
                tok_hbm.at[idx_v.at[pl.ds(b * s_per_w + c * CH, CH)]],
                tok[k],
                gsem[k],
            )

        def start_pos(c):
            return pltpu.async_copy(
                pos_hbm.at[pl.ds(s_base + c * CH, CH)],
                posb[c % NPOS],
                psem[c % NPOS],
            )

        pos_h = {c: start_pos(c) for c in range(NPOS)}
        g_h = {t: start_gather(t) for t in range(NBUF - 1)}
        out_h = {}

        for t in range(NSTEPS):
            c, b = divmod(t, B)
            if t + NBUF - 1 < NSTEPS:
                if t - 1 >= 0:
                    out_h[t - 1].wait()
                g_h[t + NBUF - 1] = start_gather(t + NBUF - 1)
            if b == 0:
                pos_h[c].wait()
            g_h[t].wait()

            tv, pv = tok[t % NBUF], posb[c % NPOS]

            def add_row(r, _, tv=tv, pv=pv):
                for j in range(E // L):
                    sl = pl.ds(j * L, L)
                    tv[r, sl] = tv[r, sl] + pv[r, sl]
                return 0

            lax.fori_loop(0, CH, add_row, 0)

            out_h[t] = pltpu.async_copy(
                tv, out_hbm.at[b, pl.ds(s_base + c * CH, CH)], osem[t % NBUF]
            )
            if b == B - 1 and c + NPOS < NCH:
                pos_h[c + NPOS] = start_pos(c + NPOS)

        for t in range(max(0, NSTEPS - NBUF), NSTEPS):
            out_h[t].wait()

    return emb_kernel


def kernel(x, tok_table, pos_table):
    B, S = x.shape
    V, E = tok_table.shape
    emb = _build_kernel(B, S, V, E)
    return emb(x.astype(jnp.int32), tok_table, pos_table)


# speedup vs baseline: 1.3755x; 1.1405x over previous
import functools

import jax
import jax.numpy as jnp
from jax import lax
from jax.experimental import pallas as pl
from jax.experimental.pallas import tpu as pltpu
from jax.experimental.pallas import tpu_sc as plsc


def _build_kernel(B, S, V, E):
    info = plsc.get_sparse_core_info()
    NC, NS, L = info.num_cores, info.num_subcores, info.num_lanes
    NW = NC * NS
    assert S % NW == 0
    s_per_w = S // NW
    CH = min(8, s_per_w)
    assert s_per_w % CH == 0 and E % L == 0
    NCH = s_per_w // CH
    NGEN = min(3, NCH)
    NPOS = min(2, NCH)

    mesh = plsc.VectorSubcoreMesh(core_axis_name="c", subcore_axis_name="s")

    scratch = [pltpu.VMEM((B * s_per_w,), jnp.int32)]
    scratch += [pltpu.VMEM((CH, E), jnp.float32) for _ in range(NGEN * B)]
    scratch += [pltpu.VMEM((CH, E), jnp.float32) for _ in range(NPOS)]
    scratch += [pltpu.SemaphoreType.DMA for _ in range(2 * NGEN + NPOS)]

    @functools.partial(
        pl.kernel,
        mesh=mesh,
        out_type=jax.ShapeDtypeStruct((B, S, E), jnp.float32),
        scratch_types=scratch,
    )
    def emb_kernel(x_hbm, tok_hbm, pos_hbm, out_hbm, idx_v, *rest):
        tok = [list(rest[g * B:(g + 1) * B]) for g in range(NGEN)]
        posb = list(rest[NGEN * B:NGEN * B + NPOS])
        sems = rest[NGEN * B + NPOS:]
        gsem = list(sems[:NGEN])
        osem = list(sems[NGEN:2 * NGEN])
        psem = list(sems[2 * NGEN:])

        wid = lax.axis_index("s") * NC + lax.axis_index("c")
        s_base = wid * s_per_w

        for b in range(B):
            pltpu.sync_copy(
                x_hbm.at[b, pl.ds(s_base, s_per_w)],
                idx_v.at[pl.ds(b * s_per_w, s_per_w)],
            )

        def start_gathers(c):
            g = c % NGEN
            return [
                pltpu.async_copy(
                    tok_hbm.at[idx_v.at[pl.ds(b * s_per_w + c * CH, CH)]],
                    tok[g][b],
                    gsem[g],
                )
                for b in range(B)
            ]

        def start_pos(c):
            return pltpu.async_copy(
                pos_hbm.at[pl.ds(s_base + c * CH, CH)],
                posb[c % NPOS],
                psem[c % NPOS],
            )

        pos_h = {c: start_pos(c) for c in range(NPOS)}
        g_h = {c: start_gathers(c) for c in range(min(NGEN - 1, NCH))}
        out_h = {}
        drained = set()

        for c in range(NCH):
            g = c % NGEN
            if c + NGEN - 1 < NCH:
                if c - 1 >= 0:
                    for h in out_h[c - 1]:
                        h.wait()
                    drained.add(c - 1)
                g_h[c + NGEN - 1] = start_gathers(c + NGEN - 1)
            pos_h[c].wait()
            for h in g_h[c]:
                h.wait()

            bufs, pv_ref = tok[g], posb[c % NPOS]

            def add_row(r, _, bufs=bufs, pv_ref=pv_ref):
                for j in range(E // L):
                    sl = pl.ds(j * L, L)
                    pv = pv_ref[r, sl]
                    for b in range(B):
                        plsc.addupdate(bufs[b].at[r, sl], pv)
                return 0

            lax.fori_loop(0, CH, add_row, 0)

            out_h[c] = [
                pltpu.async_copy(
                    bufs[b], out_hbm.at[b, pl.ds(s_base + c * CH, CH)], osem[g]
                )
                for b in range(B)
            ]
            if c + NPOS < NCH:
                pos_h[c + NPOS] = start_pos(c + NPOS)

        for c in range(NCH):
            if c not in drained:
                for h in out_h[c]:
                    h.wait()

    return emb_kernel


def kernel(x, tok_table, pos_table):
    B, S = x.shape
    V, E = tok_table.shape
    emb = _build_kernel(B, S, V, E)
    return emb(x.astype(jnp.int32), tok_table, pos_table)


# speedup vs baseline: 1.3943x; 1.0136x over previous
import functools

import jax
import jax.numpy as jnp
from jax import lax
from jax.experimental import pallas as pl
from jax.experimental.pallas import tpu as pltpu
from jax.experimental.pallas import tpu_sc as plsc


def _build_kernel(B, S, V, E):
    info = plsc.get_sparse_core_info()
    NC, NS, L = info.num_cores, info.num_subcores, info.num_lanes
    NW = NC * NS
    assert S % NW == 0
    s_per_w = S // NW
    CH = min(8, s_per_w)
    assert s_per_w % CH == 0 and E % L == 0
    NCH = s_per_w // CH
    NGEN = min(3, NCH)
    NPOS = min(2, NCH)

    mesh = plsc.VectorSubcoreMesh(core_axis_name="c", subcore_axis_name="s")

    scratch = [pltpu.VMEM((B * s_per_w,), jnp.int32)]
    scratch += [pltpu.VMEM((B * CH, E), jnp.float32) for _ in range(NGEN)]
    scratch += [pltpu.VMEM((CH, E), jnp.float32) for _ in range(NPOS)]
    scratch += [pltpu.SemaphoreType.DMA for _ in range(2 * NGEN + NPOS + 1)]

    @functools.partial(
        pl.kernel,
        mesh=mesh,
        out_type=jax.ShapeDtypeStruct((B, S, E), jnp.float32),
        scratch_types=scratch,
    )
    def emb_kernel(x_hbm, tok_hbm, pos_hbm, out_hbm, idx_v, *rest):
        bufs = list(rest[:NGEN])
        posb = list(rest[NGEN:NGEN + NPOS])
        sems = rest[NGEN + NPOS:]
        gsem = list(sems[:NGEN])
        osem = list(sems[NGEN:2 * NGEN])
        psem = list(sems[2 * NGEN:2 * NGEN + NPOS])
        isem = sems[2 * NGEN + NPOS]

        wid = lax.axis_index("s") * NC + lax.axis_index("c")
        s_base = wid * s_per_w

        idx_h = [
            pltpu.async_copy(
                x_hbm.at[b, pl.ds(s_base + c * CH, CH)],
                idx_v.at[pl.ds((c * B + b) * CH, CH)],
                isem,
            )
            for c in range(NCH)
            for b in range(B)
        ]
        for h in idx_h:
            h.wait()

        def start_gather(c):
            g = c % NGEN
            return pltpu.async_copy(
                tok_hbm.at[idx_v.at[pl.ds(c * B * CH, B * CH)]],
                bufs[g],
                gsem[g],
            )

        def start_pos(c):
            return pltpu.async_copy(
                pos_hbm.at[pl.ds(s_base + c * CH, CH)],
                posb[c % NPOS],
                psem[c % NPOS],
            )

        pos_h = {c: start_pos(c) for c in range(NPOS)}
        g_h = {c: start_gather(c) for c in range(min(NGEN - 1, NCH))}
        out_h = {}
        drained = set()

        for c in range(NCH):
            g = c % NGEN
            if c + NGEN - 1 < NCH:
                if c - 1 >= 0:
                    for h in out_h[c - 1]:
                        h.wait()
                    drained.add(c - 1)
                g_h[c + NGEN - 1] = start_gather(c + NGEN - 1)
            pos_h[c].wait()
            g_h[c].wait()

            buf, pv_ref = bufs[g], posb[c % NPOS]

            @plsc.parallel_loop(0, CH)
            def add_row(r, buf=buf, pv_ref=pv_ref):
                for j in range(E // L):
                    sl = pl.ds(j * L, L)
                    pv = pv_ref[r, sl]
                    for b in range(B):
                        plsc.addupdate(buf.at[b * CH + r, sl], pv)

            out_h[c] = [
                pltpu.async_copy(
                    buf.at[pl.ds(b * CH, CH)],
                    out_hbm.at[b, pl.ds(s_base + c * CH, CH)],
                    osem[g],
                )
                for b in range(B)
            ]
            if c + NPOS < NCH:
                pos_h[c + NPOS] = start_pos(c + NPOS)

        for c in range(NCH):
            if c not in drained:
                for h in out_h[c]:
                    h.wait()

    return emb_kernel


def kernel(x, tok_table, pos_table):
    B, S = x.shape
    V, E = tok_table.shape
    emb = _build_kernel(B, S, V, E)
    return emb(x.astype(jnp.int32), tok_table, pos_table)


# speedup vs baseline: 1.3967x; 1.0017x over previous
import functools

import jax
import jax.numpy as jnp
from jax import lax
from jax.experimental import pallas as pl
from jax.experimental.pallas import tpu as pltpu
from jax.experimental.pallas import tpu_sc as plsc


def _build_kernel(B, S, V, E):
    info = plsc.get_sparse_core_info()
    NC, NS, L = info.num_cores, info.num_subcores, info.num_lanes
    NW = NC * NS
    assert S % NW == 0
    s_per_w = S // NW
    CH = min(8, s_per_w)
    assert s_per_w % CH == 0 and E % L == 0
    NCH = s_per_w // CH
    NGEN = min(3, NCH)
    NPOS = min(2, NCH)

    mesh = plsc.VectorSubcoreMesh(core_axis_name="c", subcore_axis_name="s")

    scratch = [pltpu.VMEM((B * s_per_w,), jnp.int32)]
    scratch += [pltpu.VMEM((B * CH, E), jnp.float32) for _ in range(NGEN)]
    scratch += [pltpu.VMEM((CH, E), jnp.float32) for _ in range(NPOS)]
    scratch += [pltpu.SemaphoreType.DMA for _ in range(2 * NGEN + NPOS + 1)]

    @functools.partial(
        pl.kernel,
        mesh=mesh,
        out_type=jax.ShapeDtypeStruct((B, S, E), jnp.float32),
        scratch_types=scratch,
    )
    def emb_kernel(x_hbm, tok_hbm, pos_hbm, out_hbm, idx_v, *rest):
        bufs = list(rest[:NGEN])
        posb = list(rest[NGEN:NGEN + NPOS])
        sems = rest[NGEN + NPOS:]
        gsem = list(sems[:NGEN])
        osem = list(sems[NGEN:2 * NGEN])
        psem = list(sems[2 * NGEN:2 * NGEN + NPOS])
        isem = sems[2 * NGEN + NPOS]

        wid = lax.axis_index("s") * NC + lax.axis_index("c")
        s_base = wid * s_per_w

        idx_h = [
            pltpu.async_copy(
                x_hbm.at[b, pl.ds(s_base + c * CH, CH)],
                idx_v.at[pl.ds((c * B + b) * CH, CH)],
                isem,
            )
            for c in range(NCH)
            for b in range(B)
        ]
        for h in idx_h:
            h.wait()

        def start_gather(c):
            g = c % NGEN
            return pltpu.async_copy(
                tok_hbm.at[idx_v.at[pl.ds(c * B * CH, B * CH)]],
                bufs[g],
                gsem[g],
            )

        def start_pos(c):
            return pltpu.async_copy(
                pos_hbm.at[pl.ds(s_base + c * CH, CH)],
                posb[c % NPOS],
                psem[c % NPOS],
            )

        pos_h = {c: start_pos(c) for c in range(NPOS)}
        g_h = {c: start_gather(c) for c in range(min(NGEN - 1, NCH))}
        out_h = {}
        drained = set()

        for c in range(NCH):
            g = c % NGEN
            if c + NGEN - 1 < NCH:
                if c - 1 >= 0:
                    for h in out_h[c - 1]:
                        h.wait()
                    drained.add(c - 1)
                g_h[c + NGEN - 1] = start_gather(c + NGEN - 1)
            pos_h[c].wait()
            g_h[c].wait()

            buf, pv_ref = bufs[g], posb[c % NPOS]

            @plsc.parallel_loop(0, CH)
            def add_row(r, buf=buf, pv_ref=pv_ref):
                @pl.loop(0, E // L, step=8)
                def add_groups(j0):
                    for jj in range(8):
                        sl = pl.ds((j0 + jj) * L, L)
                        pv = pv_ref[r, sl]
                        for b in range(B):
                            plsc.addupdate(buf.at[b * CH + r, sl], pv)

            out_h[c] = [
                pltpu.async_copy(
                    buf.at[pl.ds(b * CH, CH)],
                    out_hbm.at[b, pl.ds(s_base + c * CH, CH)],
                    osem[g],
                )
                for b in range(B)
            ]
            if c + NPOS < NCH:
                pos_h[c + NPOS] = start_pos(c + NPOS)

        for c in range(NCH):
            if c not in drained:
                for h in out_h[c]:
                    h.wait()

    return emb_kernel


def kernel(x, tok_table, pos_table):
    B, S = x.shape
    V, E = tok_table.shape
    emb = _build_kernel(B, S, V, E)
    return emb(x.astype(jnp.int32), tok_table, pos_table)


# speedup vs baseline: 1.5794x; 1.1308x over previous
import functools

import jax
import jax.numpy as jnp
from jax import lax
from jax.experimental import pallas as pl
from jax.experimental.pallas import tpu as pltpu
from jax.experimental.pallas import tpu_sc as plsc


def _build_kernel(B, S, V, E):
    info = plsc.get_sparse_core_info()
    NC, NS, L = info.num_cores, info.num_subcores, info.num_lanes
    NW = NC * NS
    assert S % NW == 0
    s_per_w = S // NW
    CH = min(8, s_per_w)
    assert s_per_w % CH == 0 and E % L == 0
    NCH = s_per_w // CH
    NGEN = 2 if NCH % 2 == 0 else 1

    mesh = plsc.VectorSubcoreMesh(core_axis_name="c", subcore_axis_name="s")

    scratch = [pltpu.VMEM((B, s_per_w), jnp.int32)]
    scratch += [pltpu.VMEM((B * CH, E), jnp.float32) for _ in range(NGEN)]
    scratch += [pltpu.VMEM((CH, E), jnp.float32) for _ in range(NGEN)]
    scratch += [pltpu.SemaphoreType.DMA for _ in range(3 * NGEN + 1)]

    @functools.partial(
        pl.kernel,
        mesh=mesh,
        out_type=jax.ShapeDtypeStruct((B, S, E), jnp.float32),
        scratch_types=scratch,
    )
    def emb_kernel(x_hbm, tok_hbm, pos_hbm, out_hbm, idx_v, *rest):
        bufs = list(rest[:NGEN])
        posb = list(rest[NGEN:2 * NGEN])
        sems = rest[2 * NGEN:]
        gsem = list(sems[:NGEN])
        osem = list(sems[NGEN:2 * NGEN])
        psem = list(sems[2 * NGEN:3 * NGEN])
        isem = sems[3 * NGEN]

        wid = lax.axis_index("s") * NC + lax.axis_index("c")
        s_base = wid * s_per_w

        def gather(c, g):
            return [
                pltpu.make_async_copy(
                    tok_hbm.at[idx_v.at[b, pl.ds(c * CH, CH)]],
                    bufs[g].at[pl.ds(b * CH, CH)],
                    gsem[g],
                )
                for b in range(B)
            ]

        def pos(c, g):
            return pltpu.make_async_copy(
                pos_hbm.at[pl.ds(s_base + c * CH, CH)],
                posb[g],
                psem[g],
            )

        def out(c, g):
            return [
                pltpu.make_async_copy(
                    bufs[g].at[pl.ds(b * CH, CH)],
                    out_hbm.at[b, pl.ds(s_base + c * CH, CH)],
                    osem[g],
                )
                for b in range(B)
            ]

        ihs = [
            pltpu.make_async_copy(
                x_hbm.at[b, pl.ds(s_base, s_per_w)], idx_v.at[b], isem
            )
            for b in range(B)
        ]
        for ih in ihs:
            ih.start()
        for ih in ihs:
            ih.wait()

        for g in range(NGEN):
            pos(g, g).start()
            for d in gather(g, g):
                d.start()

        @pl.loop(0, NCH, step=NGEN)
        def step_group(c0):
            for g in range(NGEN):
                c = c0 + g
                for d in gather(c, g):
                    d.wait()
                pos(c, g).wait()

                buf, pv_ref = bufs[g], posb[g]

                @plsc.parallel_loop(0, CH)
                def add_row(r, buf=buf, pv_ref=pv_ref):
                    for j in range(E // L):
                        sl = pl.ds(j * L, L)
                        pv = pv_ref[r, sl]
                        for b in range(B):
                            plsc.addupdate(buf.at[b * CH + r, sl], pv)

                for d in out(c, g):
                    d.start()
                for d in out(c, g):
                    d.wait()

                @pl.when(c < NCH - NGEN)
                def _():
                    pos(c + NGEN, g).start()
                    for d in gather(c + NGEN, g):
                        d.start()

    return emb_kernel


def kernel(x, tok_table, pos_table):
    B, S = x.shape
    V, E = tok_table.shape
    emb = _build_kernel(B, S, V, E)
    return emb(x.astype(jnp.int32), tok_table, pos_table)
